# baseline (speedup 1.0000x reference)
import functools

import jax
import jax.numpy as jnp
from jax import lax
from jax.experimental import pallas as pl
from jax.experimental.pallas import tpu as pltpu
from jax.experimental.pallas import tpu_sc as plsc

_MAX_LEN = 2048
_D = 1024
_B = 4
_NC = 2
_NS = 16
_NW = _NC * _NS
_L_PER_W = _MAX_LEN // _NW
_CHUNK = 8
_N_CHUNKS = _L_PER_W // _CHUNK
_NGEN = 3


def _body(x_hbm, t_hbm, o_hbm, *refs):
    tbufs = list(refs[0:2])
    xbufs = [list(refs[2 + 4 * g: 6 + 4 * g]) for g in range(_NGEN)]
    tsem = refs[2 + 4 * _NGEN]
    isems = list(refs[3 + 4 * _NGEN: 3 + 5 * _NGEN])
    osems = list(refs[3 + 5 * _NGEN: 3 + 6 * _NGEN])

    wid = lax.axis_index("s") * _NC + lax.axis_index("c")
    l0 = wid * _L_PER_W

    tcp = [None] * _N_CHUNKS
    xin = [[None] * _B for _ in range(_N_CHUNKS)]
    xout = [[None] * _B for _ in range(_N_CHUNKS)]

    def rows(c):
        return pl.ds(l0 + c * _CHUNK, _CHUNK)

    def t_start(c):
        d = pltpu.make_async_copy(t_hbm.at[rows(c), :], tbufs[c % 2], tsem)
        d.start()
        tcp[c] = d

    def x_start_in(c, b):
        d = pltpu.make_async_copy(
            x_hbm.at[b, rows(c), :], xbufs[c % _NGEN][b], isems[c % _NGEN])
        d.start()
        xin[c][b] = d

    def x_start_out(c, b):
        d = pltpu.make_async_copy(
            xbufs[c % _NGEN][b], o_hbm.at[b, rows(c), :], osems[c % _NGEN])
        d.start()
        xout[c][b] = d

    def add_chunk(c):
        tref = tbufs[c % 2]
        bufs = xbufs[c % _NGEN]

        @plsc.parallel_loop(0, _CHUNK * (_D // 16), unroll=8)
        def _(i):
            r = i >> 6
            s = pl.ds((i & 63) * 16, 16)
            t = tref[r, s]
            for b in range(_B):
                bufs[b][r, s] = bufs[b][r, s] + t

    t_start(0)
    t_start(1)
    for b in range(_B):
        x_start_in(0, b)
        x_start_in(1, b)
    for c in range(_N_CHUNKS):
        tcp[c].wait()
        for b in range(_B):
            xin[c][b].wait()
        add_chunk(c)
        if c + 2 < _N_CHUNKS:
            t_start(c + 2)
        for b in range(_B):
            x_start_out(c, b)
        if c + 2 < _N_CHUNKS:
            if c - 1 >= 0:
                for b in range(_B):
                    xout[c - 1][b].wait()
            for b in range(_B):
                x_start_in(c + 2, b)
    for c in (_N_CHUNKS - 3, _N_CHUNKS - 2, _N_CHUNKS - 1):
        for b in range(_B):
            xout[c][b].wait()


@functools.partial(jax.jit, donate_argnums=())
def kernel(x, table):
    mesh = plsc.VectorSubcoreMesh(core_axis_name="c", subcore_axis_name="s")
    scratch = [pltpu.VMEM((_CHUNK, _D), jnp.float32)] * (2 + _B * _NGEN)
    scratch += [pltpu.SemaphoreType.DMA] * (1 + 2 * _NGEN)
    return pl.kernel(
        _body,
        mesh=mesh,
        out_type=jax.ShapeDtypeStruct((_B, _MAX_LEN, _D), jnp.float32),
        compiler_params=pltpu.CompilerParams(
            use_tc_tiling_on_sc=True,
            disable_bounds_checks=True,
            disable_semaphore_checks=True,
            skip_device_barrier=True,
        ),
        scratch_types=scratch,
    )(x, table)


# speedup vs baseline: 1.0139x; 1.0139x over previous
import functools

import jax
import jax.numpy as jnp
from jax import lax
from jax.experimental import pallas as pl
from jax.experimental.pallas import tpu as pltpu
from jax.experimental.pallas import tpu_sc as plsc

_MAX_LEN = 2048
_D = 1024
_B = 4
_NC = 2
_NS = 16
_NW = _NC * _NS
_L_PER_W = _MAX_LEN // _NW
_CHUNK = 8
_N_CHUNKS = _L_PER_W // _CHUNK
_NGEN = 3


def _body(x_hbm, t_hbm, o_hbm, *refs):
    tbufs = list(refs[0:2])
    xbufs = list(refs[2:2 + _NGEN])
    tsem = refs[2 + _NGEN]
    isems = list(refs[3 + _NGEN: 3 + 2 * _NGEN])
    osems = list(refs[3 + 2 * _NGEN: 3 + 3 * _NGEN])

    wid = lax.axis_index("s") * _NC + lax.axis_index("c")
    l0 = wid * _L_PER_W

    tcp = [None] * _N_CHUNKS
    xin = [None] * _N_CHUNKS
    xout = [None] * _N_CHUNKS

    def rows(c):
        return pl.ds(l0 + c * _CHUNK, _CHUNK)

    def t_start(c):
        d = pltpu.make_async_copy(t_hbm.at[rows(c), :], tbufs[c % 2], tsem)
        d.start()
        tcp[c] = d

    def x_start_in(c):
        d = pltpu.make_async_copy(
            x_hbm.at[:, rows(c), :], xbufs[c % _NGEN], isems[c % _NGEN])
        d.start()
        xin[c] = d

    def x_start_out(c):
        d = pltpu.make_async_copy(
            xbufs[c % _NGEN], o_hbm.at[:, rows(c), :], osems[c % _NGEN])
        d.start()
        xout[c] = d

    def add_chunk(c):
        tref = tbufs[c % 2]
        buf = xbufs[c % _NGEN]

        @plsc.parallel_loop(0, _CHUNK * (_D // 16), unroll=8)
        def _(i):
            r = i >> 6
            s = pl.ds((i & 63) * 16, 16)
            t = tref[r, s]
            for b in range(_B):
                buf[b, r, s] = buf[b, r, s] + t

    t_start(0)
    t_start(1)
    x_start_in(0)
    x_start_in(1)
    for c in range(_N_CHUNKS):
        tcp[c].wait()
        xin[c].wait()
        add_chunk(c)
        if c + 2 < _N_CHUNKS:
            t_start(c + 2)
        x_start_out(c)
        if c + 2 < _N_CHUNKS:
            if c - 1 >= 0:
                xout[c - 1].wait()
            x_start_in(c + 2)
    for c in (_N_CHUNKS - 3, _N_CHUNKS - 2, _N_CHUNKS - 1):
        xout[c].wait()


@functools.partial(jax.jit, donate_argnums=())
def kernel(x, table):
    mesh = plsc.VectorSubcoreMesh(core_axis_name="c", subcore_axis_name="s")
    scratch = [pltpu.VMEM((_CHUNK, _D), jnp.float32)] * 2
    scratch += [pltpu.VMEM((_B, _CHUNK, _D), jnp.float32)] * _NGEN
    scratch += [pltpu.SemaphoreType.DMA] * (1 + 2 * _NGEN)
    return pl.kernel(
        _body,
        mesh=mesh,
        out_type=jax.ShapeDtypeStruct((_B, _MAX_LEN, _D), jnp.float32),
        compiler_params=pltpu.CompilerParams(use_tc_tiling_on_sc=True),
        scratch_types=scratch,
    )(x, table)
